# baseline (speedup 1.0000x reference)
import functools

import jax
import jax.numpy as jnp
import numpy as np
from jax import lax
from jax.experimental import pallas as pl
from jax.experimental.pallas import tpu as pltpu
from jax.experimental.pallas import tpu_sc as plsc

_IN_DIM = 256
_NCLU = 512
_LAT = 256
_NTOK = 1024
_BETA = 1.0
_LAMDA = 1.0
_GAMMA = 0.01
_EPS = 1e-10
_BN_EPS = 1e-5
_TOL = 1e-4
_ITER_LIMIT = 10
_CK = 16
_CM = 128
_PTOT = _NTOK + _NCLU * (_CK - 1)
_PTOT_ALLOC = _PTOT + _CM - _CK

_F32 = jnp.float32
_I32 = jnp.int32


def _dgT(a, b):
    return lax.dot_general(a, b, (((1,), (1,)), ((), ())),
                           preferred_element_type=_F32)


def _dg(a, b):
    return lax.dot_general(a, b, (((1,), (0,)), ((), ())),
                           preferred_element_type=_F32)


def _dgL(a, b):
    return lax.dot_general(a, b, (((0,), (0,)), ((), ())),
                           preferred_element_type=_F32)


def _bnorm(h, g, bt):
    mean = jnp.sum(h, axis=0, keepdims=True) * (1.0 / _NTOK)
    c = h - mean
    var = jnp.sum(c * c, axis=0, keepdims=True) * (1.0 / _NTOK)
    return c / jnp.sqrt(var + _BN_EPS) * g + bt


def _fwd_body(refs_in, refs_out):
    (x_ref, w0, b0, w1, b1, g1, t1, w2, b2, g2, t2, w3, b3, g3, t3,
     v0, c0, e0, f0, v1, c1, e1, f1, v2, c2, e2, f2, v3, c3, clus) = refs_in
    lat_ref, rec_ref, pos_ref, counts_ref, pstart_ref = refs_out

    x = x_ref[...]
    h = jnp.maximum(_dgT(x, w0[...]) + b0[...], 0.0)
    h = _bnorm(jnp.maximum(_dgT(h, w1[...]) + b1[...], 0.0), g1[...], t1[...])
    h = _bnorm(jnp.maximum(_dgT(h, w2[...]) + b2[...], 0.0), g2[...], t2[...])
    lat = _bnorm(jnp.maximum(_dgT(h, w3[...]) + b3[...], 0.0), g3[...], t3[...])
    lat_ref[...] = lat

    d = _bnorm(jnp.maximum(_dgT(lat, v0[...]) + c0[...], 0.0), e0[...], f0[...])
    d = _bnorm(jnp.maximum(_dgT(d, v1[...]) + c1[...], 0.0), e1[...], f1[...])
    d = _bnorm(jnp.maximum(_dgT(d, v2[...]) + c2[...], 0.0), e2[...], f2[...])
    out = _dgT(d, v3[...]) + c3[...]
    r = x - out
    rec_ref[...] = jnp.sum(r * r).reshape(1, 1)

    cl = clus[...]
    g = _dgT(lat, cl)
    ones_row = jnp.ones((1, _LAT), _F32)
    cn = _dgT(ones_row, cl * cl)
    score = cn - 2.0 * g
    rowmin = jnp.min(score, axis=1, keepdims=True)
    li = lax.broadcasted_iota(_I32, (_NTOK, _NCLU), 1)
    ids = jnp.min(jnp.where(score == rowmin, li, _NCLU), axis=1, keepdims=True)

    oh = (li == ids).astype(_F32)
    ones_tok = jnp.ones((1, _NTOK), _F32)
    counts = _dg(ones_tok, oh)
    pcount = jnp.floor((counts + (_CK - 1.0)) * (1.0 / _CK)) * float(_CK)
    ri = lax.broadcasted_iota(_I32, (_NCLU, _NCLU), 0)
    ci = lax.broadcasted_iota(_I32, (_NCLU, _NCLU), 1)
    ustrict = (ri < ci).astype(_F32)
    pstart = _dg(pcount, ustrict)
    ti = lax.broadcasted_iota(_I32, (_NTOK, _NTOK), 0)
    tj = lax.broadcasted_iota(_I32, (_NTOK, _NTOK), 1)
    lstrict = (tj < ti).astype(_F32)
    prefix = _dg(lstrict, oh)
    pos = jnp.sum(oh * (pstart + prefix), axis=1, keepdims=True)
    pos_ref[...] = pos.astype(_I32)
    counts_ref[...] = counts.astype(_I32)
    pstart_ref[...] = pstart.astype(_I32)


def _fwd_call(args):
    out_shapes = (
        jax.ShapeDtypeStruct((_NTOK, _LAT), _F32),
        jax.ShapeDtypeStruct((1, 1), _F32),
        jax.ShapeDtypeStruct((_NTOK, 1), _I32),
        jax.ShapeDtypeStruct((1, _NCLU), _I32),
        jax.ShapeDtypeStruct((1, _NCLU), _I32),
    )
    def body(*refs):
        _fwd_body(refs[:len(args)], refs[len(args):])
    return pl.pallas_call(
        body,
        out_shape=out_shapes,
    )(*args)


_NC, _NS = 2, 16
_ROWS_PER_W = _NTOK // (_NC * _NS)


@functools.cache
def _make_sc_scatter():
    mesh = plsc.VectorSubcoreMesh(core_axis_name="c", subcore_axis_name="s")

    @functools.partial(
        pl.kernel,
        out_type=jax.ShapeDtypeStruct((_PTOT_ALLOC, _LAT), _F32),
        mesh=mesh,
        scratch_types=[
            pltpu.VMEM((_ROWS_PER_W,), _I32),
            pltpu.VMEM((_ROWS_PER_W, _LAT), _F32),
            pltpu.SemaphoreType.DMA,
        ],
    )
    def _sc_scatter(lat_hbm, pos_hbm, out_hbm, idx_v, rows_v, sem):
        wid = lax.axis_index("s") * _NC + lax.axis_index("c")
        base = wid * _ROWS_PER_W
        pltpu.sync_copy(pos_hbm.at[pl.ds(base, _ROWS_PER_W)], idx_v)
        pltpu.sync_copy(lat_hbm.at[pl.ds(base, _ROWS_PER_W)], rows_v)
        pltpu.async_copy(rows_v, out_hbm.at[idx_v], sem).wait()

    return _sc_scatter


def _scatter_rows(lat, pos):
    return _make_sc_scatter()(lat, pos)


def _kmeans_body(counts_ref, pstart_ref, ct_ref, xp_ref, out_ref,
                 C, prev, num, den, D, cn_ref, sc_ref, T):
    ct0 = ct_ref[...]
    C[...] = ct0
    cn_ref[...] = jnp.sum(ct0 * ct0, axis=0, keepdims=True)
    sc_ref[...] = jnp.sum(ct0, axis=0, keepdims=True)
    liota = lax.broadcasted_iota(_I32, (_LAT, _NCLU), 1)
    l512 = lax.broadcasted_iota(_I32, (1, _NCLU), 1)
    out_ref[...] = jnp.zeros((1, 1), _F32)

    def _sel(lanej, v):
        return jnp.sum(jnp.where(lanej, v, 0.0), axis=1, keepdims=True)

    def _seg_fast(M, j, cnt, ps):
        lanej = l512 == j
        vrow = lax.broadcasted_iota(_I32, (M, 1), 0) < cnt
        xc = jnp.where(vrow, xp_ref[pl.ds(ps, M), :], 0.0)
        xsq = jnp.sum(xc * xc, axis=1, keepdims=True)
        xsum = jnp.sum(xc, axis=1, keepdims=True)
        gram = _dgT(xc, xc)
        cn = cn_ref[...]
        dis0 = xsq + cn - 2.0 * _dg(xc, C[...])
        cnj0 = _sel(lanej, cn)
        discol0 = _sel(lanej, dis0)
        gj0 = (xsq + cnj0 - discol0) * 0.5
        xcv = gj0 - _EPS * xsum
        scj0 = _sel(lanej, sc_ref[...])
        vsq = cnj0 - (2.0 * _EPS) * scj0 + (_LAT * _EPS * _EPS)
        carry_t = M <= 32
        if not carry_t:
            T[pl.ds(0, M), :] = jnp.zeros((M, _NCLU), _F32)

        init = (jnp.int32(0), jnp.float32(0.0), gj0, cnj0,
                jnp.zeros((M, 1), _F32), jnp.ones((1, 1), _F32))
        if carry_t:
            init = init + (jnp.zeros((M, _NCLU), _F32),)

        def w_cond(s):
            it, shift = s[0], s[1]
            return (it == 0) | ((shift * shift >= _TOL) & (it < _ITER_LIMIT))

        def w_body(s):
            it, _, gj, cnj = s[0], s[1], s[2], s[3]
            tprev = s[6] if carry_t else T[pl.ds(0, M), :]
            dis = jnp.where(lanej, xsq + cnj - 2.0 * gj, dis0)
            mn = jnp.min(jnp.min(jnp.where(vrow, dis, jnp.inf),
                                 axis=1, keepdims=True), axis=0, keepdims=True)
            mx = jnp.max(jnp.max(jnp.where(vrow, dis, -jnp.inf),
                                 axis=1, keepdims=True), axis=0, keepdims=True)
            inv = 1.0 / (mx - mn)
            e = jnp.exp(-_GAMMA * ((dis - mn) * inv))
            rs = 1.0 / jnp.sum(e, axis=1, keepdims=True)
            soft = jnp.where(vrow, e * rs + _EPS, 0.0)
            dn = jnp.sum(soft, axis=0, keepdims=True)
            t = soft * (1.0 / dn)
            dt = t - tprev
            ssq = jnp.sum(dt * _dg(gram, dt), axis=0, keepdims=True)
            corr = vsq - 2.0 * jnp.sum(t * xcv, axis=0, keepdims=True)
            ssq = ssq + jnp.where(it == 0, corr, jnp.zeros_like(corr))
            shift = jnp.sum(jnp.sqrt(jnp.maximum(ssq, 0.0)))
            softcol = _sel(lanej, soft)
            dj = _sel(lanej, dn)
            rdj = 1.0 / dj
            gs = _dg(gram, softcol)
            nsq = jnp.sum(softcol * gs, axis=0, keepdims=True)
            sxs = jnp.sum(softcol * xsum, axis=0, keepdims=True)
            cnj_n = (nsq * rdj * rdj + (2.0 * _EPS) * sxs * rdj
                     + (_LAT * _EPS * _EPS))
            gj_n = gs * rdj + _EPS * xsum
            if not carry_t:
                T[pl.ds(0, M), :] = t
            nxt = (it + 1, shift, gj_n, cnj_n, softcol, dj)
            if carry_t:
                nxt = nxt + (t,)
            return nxt

        fin = lax.while_loop(w_cond, w_body, init)
        gjf, cnjf, softcol, dj = fin[2], fin[3], fin[4], fin[5]

        rdj = 1.0 / dj
        cjf = _dgL(xc, softcol) * rdj + _EPS
        C[...] = jnp.where(liota == j, jnp.broadcast_to(cjf, (_LAT, _NCLU)),
                           C[...])
        cn_ref[...] = jnp.where(lanej, cnjf, cn_ref[...])
        sc_ref[...] = jnp.where(lanej, jnp.sum(cjf, axis=0, keepdims=True),
                                sc_ref[...])
        contrib = jnp.where(vrow, xsq - 2.0 * gjf + cnjf, 0.0)
        out_ref[...] = out_ref[...] + jnp.sum(contrib)

    def cluster_step(j, carry):
        cnt = counts_ref[0, j]
        ps = pl.multiple_of(pstart_ref[0, j], _CK)
        nch = (cnt + (_CK - 1)) // _CK

        @pl.when((cnt > 0) & (cnt <= _CK))
        def _():
            _seg_fast(_CK, j, cnt, ps)

        @pl.when((cnt > _CK) & (cnt <= 32))
        def _():
            _seg_fast(32, j, cnt, ps)

        @pl.when((cnt > 32) & (cnt <= 64))
        def _():
            _seg_fast(64, j, cnt, ps)

        @pl.when((cnt > 64) & (cnt <= _CM))
        def _():
            _seg_fast(_CM, j, cnt, ps)

        @pl.when(cnt > _CM)
        def _():
            ohc = (lax.broadcasted_iota(_I32, (_NCLU, 1), 0) == j).astype(_F32)
            cj0 = _dg(C[...], ohc)
            prev[...] = jnp.broadcast_to(cj0, (_LAT, _NCLU))

            def w_cond(s):
                it, shift = s
                return (it == 0) | ((shift * shift >= _TOL) & (it < _ITER_LIMIT))

            def w_body(s):
                it, _ = s
                ct = C[...]
                cn = jnp.sum(ct * ct, axis=0, keepdims=True)

                def p1(c, mm):
                    mn, mx = mm
                    xc = xp_ref[pl.ds(ps + c * _CK, _CK), :]
                    vrow = (lax.broadcasted_iota(_I32, (_CK, 1), 0)
                            + c * _CK) < cnt
                    xc = jnp.where(vrow, xc, 0.0)
                    xsq = jnp.sum(xc * xc, axis=1, keepdims=True)
                    dis = xsq + cn - 2.0 * _dg(xc, ct)
                    D[pl.ds(c * _CK, _CK), :] = dis
                    mn = jnp.minimum(mn, jnp.min(jnp.where(vrow, dis, jnp.inf)))
                    mx = jnp.maximum(mx, jnp.max(jnp.where(vrow, dis, -jnp.inf)))
                    return mn, mx

                mn, mx = lax.fori_loop(0, nch, p1,
                                       (jnp.float32(np.inf),
                                        jnp.float32(-np.inf)))
                num[...] = jnp.zeros((_LAT, _NCLU), _F32)
                den[...] = jnp.zeros((1, _NCLU), _F32)
                inv = 1.0 / (mx - mn)

                def p2(c, _):
                    dis = D[pl.ds(c * _CK, _CK), :]
                    vrow = (lax.broadcasted_iota(_I32, (_CK, 1), 0)
                            + c * _CK) < cnt
                    xc = jnp.where(vrow, xp_ref[pl.ds(ps + c * _CK, _CK), :],
                                   0.0)
                    disn = (dis - mn) * inv
                    e = jnp.exp(-_GAMMA * disn)
                    rs = jnp.sum(e, axis=1, keepdims=True)
                    soft = jnp.where(vrow, e / rs + _EPS, 0.0)
                    num[...] += _dgL(xc, soft)
                    den[...] += jnp.sum(soft, axis=0, keepdims=True)
                    return 0

                lax.fori_loop(0, nch, p2, 0)
                ns = num[...] / den[...] + _EPS
                diff = ns - prev[...]
                ssq = jnp.sum(diff * diff, axis=0, keepdims=True)
                shift = jnp.sum(jnp.sqrt(ssq))
                C[...] = jnp.where(liota == j, ns, C[...])
                prev[...] = ns
                return it + 1, shift

            lax.while_loop(w_cond, w_body, (jnp.int32(0), jnp.float32(0.0)))

            cjf = _dg(C[...], ohc)
            cnj = jnp.sum(cjf * cjf)
            cn_ref[...] = jnp.where(l512 == j, cnj, cn_ref[...])
            sc_ref[...] = jnp.where(l512 == j, jnp.sum(cjf), sc_ref[...])

            def pd(c, acc):
                xc = xp_ref[pl.ds(ps + c * _CK, _CK), :]
                vrow = (lax.broadcasted_iota(_I32, (_CK, 1), 0)
                        + c * _CK) < cnt
                xc = jnp.where(vrow, xc, 0.0)
                xsq = jnp.sum(xc * xc, axis=1, keepdims=True)
                gj = _dg(xc, cjf)
                contrib = jnp.where(vrow, xsq - 2.0 * gj + cnj, 0.0)
                return acc + jnp.sum(contrib)

            dsum = lax.fori_loop(0, nch, pd, jnp.float32(0.0))
            out_ref[...] = out_ref[...] + dsum

        return carry

    lax.fori_loop(0, _NCLU, cluster_step, 0)


def _kmeans_call(counts, pstart, clusters_t, xp):
    return pl.pallas_call(
        _kmeans_body,
        out_shape=jax.ShapeDtypeStruct((1, 1), _F32),
        in_specs=[
            pl.BlockSpec(memory_space=pltpu.SMEM),
            pl.BlockSpec(memory_space=pltpu.SMEM),
            pl.BlockSpec(),
            pl.BlockSpec(),
        ],
        scratch_shapes=[
            pltpu.VMEM((_LAT, _NCLU), _F32),
            pltpu.VMEM((_LAT, _NCLU), _F32),
            pltpu.VMEM((_LAT, _NCLU), _F32),
            pltpu.VMEM((1, _NCLU), _F32),
            pltpu.VMEM((_NTOK, _NCLU), _F32),
            pltpu.VMEM((1, _NCLU), _F32),
            pltpu.VMEM((1, _NCLU), _F32),
            pltpu.VMEM((_CM, _NCLU), _F32),
        ],
    )(counts, pstart, clusters_t, xp)


def kernel(data, enc, dec, clusters):
    data2 = jnp.transpose(data, (0, 2, 1)).reshape(-1, _IN_DIM)
    r1 = lambda v: v.reshape(1, -1)
    args = (
        data2,
        enc['W0'], r1(enc['b0']),
        enc['W1'], r1(enc['b1']), r1(enc['g1']), r1(enc['bt1']),
        enc['W2'], r1(enc['b2']), r1(enc['g2']), r1(enc['bt2']),
        enc['W3'], r1(enc['b3']), r1(enc['g3']), r1(enc['bt3']),
        dec['W0'], r1(dec['b0']), r1(dec['g0']), r1(dec['bt0']),
        dec['W1'], r1(dec['b1']), r1(dec['g1']), r1(dec['bt1']),
        dec['W2'], r1(dec['b2']), r1(dec['g2']), r1(dec['bt2']),
        dec['W3'], r1(dec['b3']),
        clusters,
    )
    lat, rec_sum, pos, counts, pstart = _fwd_call(args)
    xp = _scatter_rows(lat, pos.reshape(_NTOK))
    dist_sum = _kmeans_call(counts, pstart, clusters.T, xp)
    rec_loss = rec_sum[0, 0] * (1.0 / (_NTOK * _IN_DIM))
    dist_loss = 0.5 * _BETA * dist_sum[0, 0] * (1.0 / _NTOK)
    return _LAMDA * rec_loss + _BETA * dist_loss


# speedup vs baseline: 1.0057x; 1.0057x over previous
import functools

import jax
import jax.numpy as jnp
import numpy as np
from jax import lax
from jax.experimental import pallas as pl
from jax.experimental.pallas import tpu as pltpu
from jax.experimental.pallas import tpu_sc as plsc

_IN_DIM = 256
_NCLU = 512
_LAT = 256
_NTOK = 1024
_BETA = 1.0
_LAMDA = 1.0
_GAMMA = 0.01
_EPS = 1e-10
_BN_EPS = 1e-5
_TOL = 1e-4
_ITER_LIMIT = 10
_CK = 16
_CM = 128
_PTOT = _NTOK + _NCLU * (_CK - 1)
_PTOT_ALLOC = _PTOT + _CM - _CK

_F32 = jnp.float32
_I32 = jnp.int32


def _dgT(a, b):
    return lax.dot_general(a, b, (((1,), (1,)), ((), ())),
                           preferred_element_type=_F32)


def _dg(a, b):
    return lax.dot_general(a, b, (((1,), (0,)), ((), ())),
                           preferred_element_type=_F32)


def _dgL(a, b):
    return lax.dot_general(a, b, (((0,), (0,)), ((), ())),
                           preferred_element_type=_F32)


def _bnorm(h, g, bt):
    mean = jnp.sum(h, axis=0, keepdims=True) * (1.0 / _NTOK)
    c = h - mean
    var = jnp.sum(c * c, axis=0, keepdims=True) * (1.0 / _NTOK)
    return c / jnp.sqrt(var + _BN_EPS) * g + bt


def _fwd_body(refs_in, refs_out):
    (x_ref, w0, b0, w1, b1, g1, t1, w2, b2, g2, t2, w3, b3, g3, t3,
     v0, c0, e0, f0, v1, c1, e1, f1, v2, c2, e2, f2, v3, c3, clus) = refs_in
    lat_ref, rec_ref, pos_ref, counts_ref, pstart_ref = refs_out

    x = x_ref[...]
    h = jnp.maximum(_dgT(x, w0[...]) + b0[...], 0.0)
    h = _bnorm(jnp.maximum(_dgT(h, w1[...]) + b1[...], 0.0), g1[...], t1[...])
    h = _bnorm(jnp.maximum(_dgT(h, w2[...]) + b2[...], 0.0), g2[...], t2[...])
    lat = _bnorm(jnp.maximum(_dgT(h, w3[...]) + b3[...], 0.0), g3[...], t3[...])
    lat_ref[...] = lat

    d = _bnorm(jnp.maximum(_dgT(lat, v0[...]) + c0[...], 0.0), e0[...], f0[...])
    d = _bnorm(jnp.maximum(_dgT(d, v1[...]) + c1[...], 0.0), e1[...], f1[...])
    d = _bnorm(jnp.maximum(_dgT(d, v2[...]) + c2[...], 0.0), e2[...], f2[...])
    out = _dgT(d, v3[...]) + c3[...]
    r = x - out
    rec_ref[...] = jnp.sum(r * r).reshape(1, 1)

    cl = clus[...]
    g = _dgT(lat, cl)
    ones_row = jnp.ones((1, _LAT), _F32)
    cn = _dgT(ones_row, cl * cl)
    score = cn - 2.0 * g
    rowmin = jnp.min(score, axis=1, keepdims=True)
    li = lax.broadcasted_iota(_I32, (_NTOK, _NCLU), 1)
    ids = jnp.min(jnp.where(score == rowmin, li, _NCLU), axis=1, keepdims=True)

    oh = (li == ids).astype(_F32)
    ones_tok = jnp.ones((1, _NTOK), _F32)
    counts = _dg(ones_tok, oh)
    pcount = jnp.floor((counts + (_CK - 1.0)) * (1.0 / _CK)) * float(_CK)
    ri = lax.broadcasted_iota(_I32, (_NCLU, _NCLU), 0)
    ci = lax.broadcasted_iota(_I32, (_NCLU, _NCLU), 1)
    ustrict = (ri < ci).astype(_F32)
    pstart = _dg(pcount, ustrict)
    ti = lax.broadcasted_iota(_I32, (_NTOK, _NTOK), 0)
    tj = lax.broadcasted_iota(_I32, (_NTOK, _NTOK), 1)
    lstrict = (tj < ti).astype(_F32)
    prefix = _dg(lstrict, oh)
    pos = jnp.sum(oh * (pstart + prefix), axis=1, keepdims=True)
    pos_ref[...] = pos.astype(_I32)
    counts_ref[...] = counts.astype(_I32)
    pstart_ref[...] = pstart.astype(_I32)


def _fwd_call(args):
    out_shapes = (
        jax.ShapeDtypeStruct((_NTOK, _LAT), _F32),
        jax.ShapeDtypeStruct((1, 1), _F32),
        jax.ShapeDtypeStruct((_NTOK, 1), _I32),
        jax.ShapeDtypeStruct((1, _NCLU), _I32),
        jax.ShapeDtypeStruct((1, _NCLU), _I32),
    )
    def body(*refs):
        _fwd_body(refs[:len(args)], refs[len(args):])
    return pl.pallas_call(
        body,
        out_shape=out_shapes,
    )(*args)


_NC, _NS = 2, 16
_ROWS_PER_W = _NTOK // (_NC * _NS)


@functools.cache
def _make_sc_scatter():
    mesh = plsc.VectorSubcoreMesh(core_axis_name="c", subcore_axis_name="s")

    @functools.partial(
        pl.kernel,
        out_type=jax.ShapeDtypeStruct((_PTOT_ALLOC, _LAT), _F32),
        mesh=mesh,
        scratch_types=[
            pltpu.VMEM((_ROWS_PER_W,), _I32),
            pltpu.VMEM((_ROWS_PER_W, _LAT), _F32),
            pltpu.SemaphoreType.DMA,
        ],
    )
    def _sc_scatter(lat_hbm, pos_hbm, out_hbm, idx_v, rows_v, sem):
        wid = lax.axis_index("s") * _NC + lax.axis_index("c")
        base = wid * _ROWS_PER_W
        pltpu.sync_copy(pos_hbm.at[pl.ds(base, _ROWS_PER_W)], idx_v)
        pltpu.sync_copy(lat_hbm.at[pl.ds(base, _ROWS_PER_W)], rows_v)
        pltpu.async_copy(rows_v, out_hbm.at[idx_v], sem).wait()

    return _sc_scatter


def _scatter_rows(lat, pos):
    return _make_sc_scatter()(lat, pos)


def _kmeans_body(counts_ref, pstart_ref, ct_ref, xp_ref, out_ref,
                 C, prev, num, den, D, cn_ref, sc_ref, T):
    ct0 = ct_ref[...]
    C[...] = ct0
    cn_ref[...] = jnp.sum(ct0 * ct0, axis=0, keepdims=True)
    sc_ref[...] = jnp.sum(ct0, axis=0, keepdims=True)
    liota = lax.broadcasted_iota(_I32, (_LAT, _NCLU), 1)
    l512 = lax.broadcasted_iota(_I32, (1, _NCLU), 1)
    out_ref[...] = jnp.zeros((1, 1), _F32)

    def _sel(lanej, v):
        return jnp.sum(jnp.where(lanej, v, 0.0), axis=1, keepdims=True)

    def _seg_fast(M, j, cnt, ps):
        lanej = l512 == j
        vrow = lax.broadcasted_iota(_I32, (M, 1), 0) < cnt
        xc = jnp.where(vrow, xp_ref[pl.ds(ps, M), :], 0.0)
        xsq = jnp.sum(xc * xc, axis=1, keepdims=True)
        xsum = jnp.sum(xc, axis=1, keepdims=True)
        gram = _dgT(xc, xc)
        cn = cn_ref[...]
        dis0 = xsq + cn - 2.0 * _dg(xc, C[...])
        cnj0 = _sel(lanej, cn)
        discol0 = _sel(lanej, dis0)
        gj0 = (xsq + cnj0 - discol0) * 0.5
        xcv = gj0 - _EPS * xsum
        scj0 = _sel(lanej, sc_ref[...])
        vsq = cnj0 - (2.0 * _EPS) * scj0 + (_LAT * _EPS * _EPS)
        carry_t = M <= 32
        if not carry_t:
            T[pl.ds(0, M), :] = jnp.zeros((M, _NCLU), _F32)

        init = (jnp.int32(0), jnp.float32(0.0), gj0, cnj0,
                jnp.zeros((M, 1), _F32), jnp.ones((1, 1), _F32))
        if carry_t:
            init = init + (jnp.zeros((M, _NCLU), _F32),)

        def w_cond(s):
            it, shift = s[0], s[1]
            return (it == 0) | ((shift * shift >= _TOL) & (it < _ITER_LIMIT))

        def w_body(s):
            it, _, gj, cnj = s[0], s[1], s[2], s[3]
            tprev = s[6] if carry_t else T[pl.ds(0, M), :]
            dis = jnp.where(lanej, xsq + cnj - 2.0 * gj, dis0)
            mn = jnp.min(jnp.min(jnp.where(vrow, dis, jnp.inf),
                                 axis=0, keepdims=True), axis=1, keepdims=True)
            mx = jnp.max(jnp.max(jnp.where(vrow, dis, -jnp.inf),
                                 axis=0, keepdims=True), axis=1, keepdims=True)
            inv = 1.0 / (mx - mn)
            x = -_GAMMA * ((dis - mn) * inv)
            e = 1.0 + x * (1.0 + x * (0.5 + x * (1.0 / 6.0)))
            rs = 1.0 / jnp.sum(e, axis=1, keepdims=True)
            soft = jnp.where(vrow, e * rs + _EPS, 0.0)
            dn = jnp.sum(soft, axis=0, keepdims=True)
            t = soft * (1.0 / dn)
            dt = t - tprev
            ssq = jnp.sum(dt * _dg(gram, dt), axis=0, keepdims=True)
            corr = vsq - 2.0 * jnp.sum(t * xcv, axis=0, keepdims=True)
            ssq = ssq + jnp.where(it == 0, corr, jnp.zeros_like(corr))
            shift = jnp.sum(jnp.sqrt(jnp.maximum(ssq, 0.0)))
            softcol = _sel(lanej, soft)
            dj = _sel(lanej, dn)
            rdj = 1.0 / dj
            gs = _dg(gram, softcol)
            nsq = jnp.sum(softcol * gs, axis=0, keepdims=True)
            sxs = jnp.sum(softcol * xsum, axis=0, keepdims=True)
            cnj_n = (nsq * rdj * rdj + (2.0 * _EPS) * sxs * rdj
                     + (_LAT * _EPS * _EPS))
            gj_n = gs * rdj + _EPS * xsum
            if not carry_t:
                T[pl.ds(0, M), :] = t
            nxt = (it + 1, shift, gj_n, cnj_n, softcol, dj)
            if carry_t:
                nxt = nxt + (t,)
            return nxt

        fin = lax.while_loop(w_cond, w_body, init)
        gjf, cnjf, softcol, dj = fin[2], fin[3], fin[4], fin[5]

        rdj = 1.0 / dj
        cjf = _dgL(xc, softcol) * rdj + _EPS
        C[...] = jnp.where(liota == j, jnp.broadcast_to(cjf, (_LAT, _NCLU)),
                           C[...])
        cn_ref[...] = jnp.where(lanej, cnjf, cn_ref[...])
        sc_ref[...] = jnp.where(lanej, jnp.sum(cjf, axis=0, keepdims=True),
                                sc_ref[...])
        contrib = jnp.where(vrow, xsq - 2.0 * gjf + cnjf, 0.0)
        out_ref[...] = out_ref[...] + jnp.sum(contrib)

    def cluster_step(j, carry):
        cnt = counts_ref[0, j]
        ps = pl.multiple_of(pstart_ref[0, j], _CK)
        nch = (cnt + (_CK - 1)) // _CK

        @pl.when((cnt > 0) & (cnt <= _CK))
        def _():
            _seg_fast(_CK, j, cnt, ps)

        @pl.when((cnt > _CK) & (cnt <= 32))
        def _():
            _seg_fast(32, j, cnt, ps)

        @pl.when((cnt > 32) & (cnt <= 64))
        def _():
            _seg_fast(64, j, cnt, ps)

        @pl.when((cnt > 64) & (cnt <= _CM))
        def _():
            _seg_fast(_CM, j, cnt, ps)

        @pl.when(cnt > _CM)
        def _():
            ohc = (lax.broadcasted_iota(_I32, (_NCLU, 1), 0) == j).astype(_F32)
            cj0 = _dg(C[...], ohc)
            prev[...] = jnp.broadcast_to(cj0, (_LAT, _NCLU))

            def w_cond(s):
                it, shift = s
                return (it == 0) | ((shift * shift >= _TOL) & (it < _ITER_LIMIT))

            def w_body(s):
                it, _ = s
                ct = C[...]
                cn = jnp.sum(ct * ct, axis=0, keepdims=True)

                def p1(c, mm):
                    mn, mx = mm
                    xc = xp_ref[pl.ds(ps + c * _CK, _CK), :]
                    vrow = (lax.broadcasted_iota(_I32, (_CK, 1), 0)
                            + c * _CK) < cnt
                    xc = jnp.where(vrow, xc, 0.0)
                    xsq = jnp.sum(xc * xc, axis=1, keepdims=True)
                    dis = xsq + cn - 2.0 * _dg(xc, ct)
                    D[pl.ds(c * _CK, _CK), :] = dis
                    mn = jnp.minimum(mn, jnp.min(jnp.where(vrow, dis, jnp.inf)))
                    mx = jnp.maximum(mx, jnp.max(jnp.where(vrow, dis, -jnp.inf)))
                    return mn, mx

                mn, mx = lax.fori_loop(0, nch, p1,
                                       (jnp.float32(np.inf),
                                        jnp.float32(-np.inf)))
                num[...] = jnp.zeros((_LAT, _NCLU), _F32)
                den[...] = jnp.zeros((1, _NCLU), _F32)
                inv = 1.0 / (mx - mn)

                def p2(c, _):
                    dis = D[pl.ds(c * _CK, _CK), :]
                    vrow = (lax.broadcasted_iota(_I32, (_CK, 1), 0)
                            + c * _CK) < cnt
                    xc = jnp.where(vrow, xp_ref[pl.ds(ps + c * _CK, _CK), :],
                                   0.0)
                    disn = (dis - mn) * inv
                    e = jnp.exp(-_GAMMA * disn)
                    rs = jnp.sum(e, axis=1, keepdims=True)
                    soft = jnp.where(vrow, e / rs + _EPS, 0.0)
                    num[...] += _dgL(xc, soft)
                    den[...] += jnp.sum(soft, axis=0, keepdims=True)
                    return 0

                lax.fori_loop(0, nch, p2, 0)
                ns = num[...] / den[...] + _EPS
                diff = ns - prev[...]
                ssq = jnp.sum(diff * diff, axis=0, keepdims=True)
                shift = jnp.sum(jnp.sqrt(ssq))
                C[...] = jnp.where(liota == j, ns, C[...])
                prev[...] = ns
                return it + 1, shift

            lax.while_loop(w_cond, w_body, (jnp.int32(0), jnp.float32(0.0)))

            cjf = _dg(C[...], ohc)
            cnj = jnp.sum(cjf * cjf)
            cn_ref[...] = jnp.where(l512 == j, cnj, cn_ref[...])
            sc_ref[...] = jnp.where(l512 == j, jnp.sum(cjf), sc_ref[...])

            def pd(c, acc):
                xc = xp_ref[pl.ds(ps + c * _CK, _CK), :]
                vrow = (lax.broadcasted_iota(_I32, (_CK, 1), 0)
                        + c * _CK) < cnt
                xc = jnp.where(vrow, xc, 0.0)
                xsq = jnp.sum(xc * xc, axis=1, keepdims=True)
                gj = _dg(xc, cjf)
                contrib = jnp.where(vrow, xsq - 2.0 * gj + cnj, 0.0)
                return acc + jnp.sum(contrib)

            dsum = lax.fori_loop(0, nch, pd, jnp.float32(0.0))
            out_ref[...] = out_ref[...] + dsum

        return carry

    lax.fori_loop(0, _NCLU, cluster_step, 0)


def _kmeans_call(counts, pstart, clusters_t, xp):
    return pl.pallas_call(
        _kmeans_body,
        out_shape=jax.ShapeDtypeStruct((1, 1), _F32),
        in_specs=[
            pl.BlockSpec(memory_space=pltpu.SMEM),
            pl.BlockSpec(memory_space=pltpu.SMEM),
            pl.BlockSpec(),
            pl.BlockSpec(),
        ],
        scratch_shapes=[
            pltpu.VMEM((_LAT, _NCLU), _F32),
            pltpu.VMEM((_LAT, _NCLU), _F32),
            pltpu.VMEM((_LAT, _NCLU), _F32),
            pltpu.VMEM((1, _NCLU), _F32),
            pltpu.VMEM((_NTOK, _NCLU), _F32),
            pltpu.VMEM((1, _NCLU), _F32),
            pltpu.VMEM((1, _NCLU), _F32),
            pltpu.VMEM((_CM, _NCLU), _F32),
        ],
    )(counts, pstart, clusters_t, xp)


def kernel(data, enc, dec, clusters):
    data2 = jnp.transpose(data, (0, 2, 1)).reshape(-1, _IN_DIM)
    r1 = lambda v: v.reshape(1, -1)
    args = (
        data2,
        enc['W0'], r1(enc['b0']),
        enc['W1'], r1(enc['b1']), r1(enc['g1']), r1(enc['bt1']),
        enc['W2'], r1(enc['b2']), r1(enc['g2']), r1(enc['bt2']),
        enc['W3'], r1(enc['b3']), r1(enc['g3']), r1(enc['bt3']),
        dec['W0'], r1(dec['b0']), r1(dec['g0']), r1(dec['bt0']),
        dec['W1'], r1(dec['b1']), r1(dec['g1']), r1(dec['bt1']),
        dec['W2'], r1(dec['b2']), r1(dec['g2']), r1(dec['bt2']),
        dec['W3'], r1(dec['b3']),
        clusters,
    )
    lat, rec_sum, pos, counts, pstart = _fwd_call(args)
    xp = _scatter_rows(lat, pos.reshape(_NTOK))
    dist_sum = _kmeans_call(counts, pstart, clusters.T, xp)
    rec_loss = rec_sum[0, 0] * (1.0 / (_NTOK * _IN_DIM))
    dist_loss = 0.5 * _BETA * dist_sum[0, 0] * (1.0 / _NTOK)
    return _LAMDA * rec_loss + _BETA * dist_loss
